# baseline (speedup 1.0000x reference)
from functools import partial

import jax
import jax.numpy as jnp
from jax.experimental import pallas as pl
from jax.experimental.pallas import tpu as pltpu

N = 1024
D_INV = 64
N_VEC = 16
NUM_BASIS = 8
RB = 128
SB = 256
N_RB = N // RB
N_SB = N // SB

_INTERPRET = False


def _mm(a, b):
    return jax.lax.dot_general(
        a, b, (((1,), (0,)), ((), ())),
        preferred_element_type=jnp.float32)


def _layer_kernel(*refs, head):
    if head:
        (xr, xT, xa, hf, hr, hv, W1, b1, W2, b2, W3i, W3v,
         Wua, Wub, Wuc, bu, Wma, Wmb, Woi, boi, Wov, oif, ovf) = refs
    else:
        (xr, xT, xa, hf, hr, hv, W1, b1, W2, b2, W3i, W3v,
         Wua, Wub, Wuc, bu, Wma, Wmb, oh, ohv) = refs

    r = pl.program_id(0)
    centers = jax.lax.broadcasted_iota(
        jnp.int32, (1, 1, NUM_BASIS), 2).astype(jnp.float32) * (3.0 / 7.0)
    centers_t = jax.lax.broadcasted_iota(
        jnp.int32, (1, NUM_BASIS, 1), 1).astype(jnp.float32) * (3.0 / 7.0)
    def silu(v):
        return v * (0.5 * jnp.tanh(0.5 * v) + 0.5)

    def body(si, carry):
        acc_i, acc_t = carry
        s0 = si * SB
        xs = xT[:, pl.ds(s0, SB)]
        dx0 = xr[:, 0:1] - xs[0:1, :]
        dx1 = xr[:, 1:2] - xs[1:2, :]
        dx2 = xr[:, 2:3] - xs[2:3, :]
        d2 = dx0 * dx0 + dx1 * dx1 + dx2 * dx2 + 1e-8
        dinv = jax.lax.rsqrt(d2)
        dist = d2 * dinv
        rows = jax.lax.broadcasted_iota(jnp.int32, (RB, SB), 0) + r * RB
        cols = jax.lax.broadcasted_iota(jnp.int32, (RB, SB), 1) + s0
        dinv_od = jnp.where(rows == cols, 0.0, dinv)
        rbf_t = jnp.exp(-2.0 * jnp.square(dist[:, None, :] - centers_t))
        hid = jax.lax.dot_general(
            rbf_t, W1[...], (((1,), (0,)), ((), ())),
            preferred_element_type=jnp.float32)
        hid = silu(hid.reshape(RB * SB, 64) + b1[...])
        hid = silu(_mm(hid, W2[...]) + b2[...])
        ewi = _mm(hid, W3i[...]).reshape(RB, SB, D_INV)
        ewv = _mm(hid, W3v[...]).reshape(RB, SB, N_VEC)
        hs = hf[pl.ds(s0, SB), :]
        acc_i = acc_i + jnp.sum(ewi * hs[None, :, :], axis=1)
        p = jnp.transpose(ewv, (0, 2, 1)) * dinv_od[:, None, :]
        acc_t = acc_t + _mm(p.reshape(RB * N_VEC, SB), xa[pl.ds(s0, SB), :])
        return acc_i, acc_t

    carry = (jnp.zeros((RB, D_INV), jnp.float32),
             jnp.zeros((RB * N_VEC, 4), jnp.float32))
    for si in range(N_SB):
        carry = body(si, carry)
    acc_i, acc_t = carry
    acc_t = acc_t.reshape(RB, N_VEC, 4)
    s1 = acc_t[:, :, 3]
    av0 = xr[:, 0:1] * s1 - acc_t[:, :, 0]
    av1 = xr[:, 1:2] * s1 - acc_t[:, :, 1]
    av2 = xr[:, 2:3] * s1 - acc_t[:, :, 2]

    h_r = hr[...]
    d0 = 1e-4
    rbf0 = jnp.exp(-2.0 * jnp.square(
        jnp.full((1, 1, NUM_BASIS), d0, jnp.float32) - centers)).reshape(1, NUM_BASIS)
    hid0 = silu(_mm(rbf0, W1[...]) + b1[...])
    hid0 = silu(_mm(hid0, W2[...]) + b2[...])
    w0 = _mm(hid0, W3i[...])
    acc_i = acc_i - w0 * h_r

    scale = 1.0 / float(N)
    agg_i = acc_i * scale
    av0 = av0 * scale
    av1 = av1 * scale
    av2 = av2 * scale
    vec_norm = jnp.sqrt(av0 * av0 + av1 * av1 + av2 * av2 + 1e-8)

    upd = _mm(h_r, Wua[...]) + _mm(agg_i, Wub[...]) + _mm(vec_norm, Wuc[...]) + bu[...]
    h_new = h_r + silu(upd)

    hv_new = []
    for d, av in enumerate((av0, av1, av2)):
        hv_new.append(_mm(hv[d], Wma[...]) + _mm(av, Wmb[...]))

    if head:
        oif[...] = _mm(h_new, Woi[...]) + boi[...]
        for d in range(3):
            ovf[d] = _mm(hv_new[d], Wov[...])
    else:
        oh[...] = h_new
        for d in range(3):
            ohv[d] = hv_new[d]


def _full(shape):
    nd = len(shape)
    return pl.BlockSpec(shape, lambda r, _n=nd: (0,) * _n)


def _layer_call(head):
    in_specs = [
        pl.BlockSpec((RB, 3), lambda r: (r, 0)),
        _full((3, N)),
        _full((N, 4)),
        _full((N, D_INV)),
        pl.BlockSpec((RB, D_INV), lambda r: (r, 0)),
        pl.BlockSpec((3, RB, N_VEC), lambda r: (0, r, 0)),
        _full((NUM_BASIS, 64)), _full((1, 64)),
        _full((64, 64)), _full((1, 64)),
        _full((64, D_INV)), _full((64, N_VEC)),
        _full((D_INV, D_INV)), _full((D_INV, D_INV)), _full((N_VEC, D_INV)),
        _full((1, D_INV)),
        _full((N_VEC, N_VEC)), _full((N_VEC, N_VEC)),
    ]
    if head:
        in_specs += [_full((D_INV, 64)), _full((1, 64)), _full((N_VEC, 16))]
        out_specs = [
            pl.BlockSpec((RB, 64), lambda r: (r, 0)),
            pl.BlockSpec((3, RB, 16), lambda r: (0, r, 0)),
        ]
        out_shape = [
            jax.ShapeDtypeStruct((N, 64), jnp.float32),
            jax.ShapeDtypeStruct((3, N, 16), jnp.float32),
        ]
    else:
        out_specs = [
            pl.BlockSpec((RB, D_INV), lambda r: (r, 0)),
            pl.BlockSpec((3, RB, N_VEC), lambda r: (0, r, 0)),
        ]
        out_shape = [
            jax.ShapeDtypeStruct((N, D_INV), jnp.float32),
            jax.ShapeDtypeStruct((3, N, N_VEC), jnp.float32),
        ]
    return pl.pallas_call(
        partial(_layer_kernel, head=head),
        grid=(N_RB,),
        in_specs=in_specs,
        out_specs=out_specs,
        out_shape=out_shape,
        compiler_params=pltpu.CompilerParams(
            dimension_semantics=("parallel",)),
        interpret=_INTERPRET,
    )


def kernel(x, h, species_embed,
           W_r1_0, b_r1_0, W_r2_0, b_r2_0, W_r3_0, W_upd_0, b_upd_0, W_vecmix_0,
           W_r1_1, b_r1_1, W_r2_1, b_r2_1, W_r3_1, W_upd_1, b_upd_1, W_vecmix_1,
           W_out_inv, b_out_inv, W_out_vec):
    x = x.astype(jnp.float32)
    xT = x.T
    xa = jnp.concatenate([x, jnp.ones((N, 1), jnp.float32)], axis=1)
    h0 = species_embed[h]
    hv0 = jnp.zeros((3, N, N_VEC), jnp.float32)

    def layer_args(W_r1, b_r1, W_r2, b_r2, W_r3, W_upd, b_upd, W_vecmix):
        return (W_r1, b_r1.reshape(1, -1), W_r2, b_r2.reshape(1, -1),
                W_r3[:, :D_INV], W_r3[:, D_INV:],
                W_upd[:D_INV], W_upd[D_INV:2 * D_INV], W_upd[2 * D_INV:],
                b_upd.reshape(1, -1),
                W_vecmix[:N_VEC], W_vecmix[N_VEC:])

    h1, hv1 = _layer_call(False)(
        x, xT, xa, h0, h0, hv0,
        *layer_args(W_r1_0, b_r1_0, W_r2_0, b_r2_0, W_r3_0, W_upd_0, b_upd_0, W_vecmix_0))

    invf, vfT = _layer_call(True)(
        x, xT, xa, h1, h1, hv1,
        *layer_args(W_r1_1, b_r1_1, W_r2_1, b_r2_1, W_r3_1, W_upd_1, b_upd_1, W_vecmix_1),
        W_out_inv, b_out_inv.reshape(1, -1), W_out_vec)

    vector_features = jnp.transpose(vfT, (1, 2, 0))
    return vector_features, invf


# speedup vs baseline: 1.2049x; 1.2049x over previous
from functools import partial

import jax
import jax.numpy as jnp
from jax.experimental import pallas as pl
from jax.experimental.pallas import tpu as pltpu

N = 1024
D_INV = 64
N_VEC = 16
NUM_BASIS = 8
RB = 64
SB = 256
N_RB = N // RB
N_SB = N // SB


def _mm(a, b):
    return jax.lax.dot_general(
        a, b, (((1,), (0,)), ((), ())),
        preferred_element_type=jnp.float32)


def _layer_kernel(*refs, head):
    if head:
        (xr, xT, xa, hf, hr, hv, W1, b1, W2, b2, W3i, W3v,
         Wua, Wub, Wuc, bu, Wma, Wmb, Woi, boi, Wov, oif, ovf) = refs
    else:
        (xr, xT, xa, hf, hr, hv, W1, b1, W2, b2, W3i, W3v,
         Wua, Wub, Wuc, bu, Wma, Wmb, oh, ohv) = refs

    r = pl.program_id(0)
    centers = jax.lax.broadcasted_iota(
        jnp.int32, (1, 1, NUM_BASIS), 2).astype(jnp.float32) * (3.0 / 7.0)
    centers_t = jax.lax.broadcasted_iota(
        jnp.int32, (1, NUM_BASIS, 1), 1).astype(jnp.float32) * (3.0 / 7.0)
    def silu(v):
        return v * (0.5 * jnp.tanh(0.5 * v) + 0.5)

    def body(si, carry):
        acc_i, acc_t = carry
        s0 = si * SB
        xs = xT[:, pl.ds(s0, SB)]
        dx0 = xr[:, 0:1] - xs[0:1, :]
        dx1 = xr[:, 1:2] - xs[1:2, :]
        dx2 = xr[:, 2:3] - xs[2:3, :]
        d2 = dx0 * dx0 + dx1 * dx1 + dx2 * dx2 + 1e-8
        dinv = jax.lax.rsqrt(d2)
        dist = d2 * dinv
        rows = jax.lax.broadcasted_iota(jnp.int32, (RB, SB), 0) + r * RB
        cols = jax.lax.broadcasted_iota(jnp.int32, (RB, SB), 1) + s0
        dinv_od = jnp.where(rows == cols, 0.0, dinv)
        rbf_t = jnp.exp(-2.0 * jnp.square(dist[:, None, :] - centers_t))
        hid = jax.lax.dot_general(
            rbf_t, W1[...], (((1,), (0,)), ((), ())),
            preferred_element_type=jnp.float32)
        hid = silu(hid.reshape(RB * SB, 64) + b1[...])
        hid = silu(_mm(hid, W2[...]) + b2[...])
        ewi = _mm(hid, W3i[...]).reshape(RB, SB, D_INV)
        ewv = _mm(hid, W3v[...]).reshape(RB, SB, N_VEC)
        hs = hf[pl.ds(s0, SB), :]
        acc_i = acc_i + jnp.sum(ewi * hs[None, :, :], axis=1)
        p = jnp.transpose(ewv, (0, 2, 1)) * dinv_od[:, None, :]
        acc_t = acc_t + _mm(p.reshape(RB * N_VEC, SB), xa[pl.ds(s0, SB), :])
        return acc_i, acc_t

    carry = (jnp.zeros((RB, D_INV), jnp.float32),
             jnp.zeros((RB * N_VEC, 4), jnp.float32))
    for si in range(N_SB):
        carry = body(si, carry)
    acc_i, acc_t = carry
    acc_t = acc_t.reshape(RB, N_VEC, 4)
    s1 = acc_t[:, :, 3]
    av0 = xr[:, 0:1] * s1 - acc_t[:, :, 0]
    av1 = xr[:, 1:2] * s1 - acc_t[:, :, 1]
    av2 = xr[:, 2:3] * s1 - acc_t[:, :, 2]

    h_r = hr[...]
    d0 = 1e-4
    rbf0 = jnp.exp(-2.0 * jnp.square(
        jnp.full((1, 1, NUM_BASIS), d0, jnp.float32) - centers)).reshape(1, NUM_BASIS)
    hid0 = silu(_mm(rbf0, W1[...]) + b1[...])
    hid0 = silu(_mm(hid0, W2[...]) + b2[...])
    w0 = _mm(hid0, W3i[...])
    acc_i = acc_i - w0 * h_r

    scale = 1.0 / float(N)
    agg_i = acc_i * scale
    av0 = av0 * scale
    av1 = av1 * scale
    av2 = av2 * scale
    vec_norm = jnp.sqrt(av0 * av0 + av1 * av1 + av2 * av2 + 1e-8)

    upd = _mm(h_r, Wua[...]) + _mm(agg_i, Wub[...]) + _mm(vec_norm, Wuc[...]) + bu[...]
    h_new = h_r + silu(upd)

    hv_new = []
    for d, av in enumerate((av0, av1, av2)):
        hv_new.append(_mm(hv[d], Wma[...]) + _mm(av, Wmb[...]))

    if head:
        oif[...] = _mm(h_new, Woi[...]) + boi[...]
        for d in range(3):
            ovf[d] = _mm(hv_new[d], Wov[...])
    else:
        oh[...] = h_new
        for d in range(3):
            ohv[d] = hv_new[d]


def _full(shape):
    nd = len(shape)
    return pl.BlockSpec(shape, lambda r, _n=nd: (0,) * _n)


def _layer_call(head):
    in_specs = [
        pl.BlockSpec((RB, 3), lambda r: (r, 0)),
        _full((3, N)),
        _full((N, 4)),
        _full((N, D_INV)),
        pl.BlockSpec((RB, D_INV), lambda r: (r, 0)),
        pl.BlockSpec((3, RB, N_VEC), lambda r: (0, r, 0)),
        _full((NUM_BASIS, 64)), _full((1, 64)),
        _full((64, 64)), _full((1, 64)),
        _full((64, D_INV)), _full((64, N_VEC)),
        _full((D_INV, D_INV)), _full((D_INV, D_INV)), _full((N_VEC, D_INV)),
        _full((1, D_INV)),
        _full((N_VEC, N_VEC)), _full((N_VEC, N_VEC)),
    ]
    if head:
        in_specs += [_full((D_INV, 64)), _full((1, 64)), _full((N_VEC, 16))]
        out_specs = [
            pl.BlockSpec((RB, 64), lambda r: (r, 0)),
            pl.BlockSpec((3, RB, 16), lambda r: (0, r, 0)),
        ]
        out_shape = [
            jax.ShapeDtypeStruct((N, 64), jnp.float32),
            jax.ShapeDtypeStruct((3, N, 16), jnp.float32),
        ]
    else:
        out_specs = [
            pl.BlockSpec((RB, D_INV), lambda r: (r, 0)),
            pl.BlockSpec((3, RB, N_VEC), lambda r: (0, r, 0)),
        ]
        out_shape = [
            jax.ShapeDtypeStruct((N, D_INV), jnp.float32),
            jax.ShapeDtypeStruct((3, N, N_VEC), jnp.float32),
        ]
    return pl.pallas_call(
        partial(_layer_kernel, head=head),
        grid=(N_RB,),
        in_specs=in_specs,
        out_specs=out_specs,
        out_shape=out_shape,
        compiler_params=pltpu.CompilerParams(
            dimension_semantics=("parallel",)),
    )


def kernel(x, h, species_embed,
           W_r1_0, b_r1_0, W_r2_0, b_r2_0, W_r3_0, W_upd_0, b_upd_0, W_vecmix_0,
           W_r1_1, b_r1_1, W_r2_1, b_r2_1, W_r3_1, W_upd_1, b_upd_1, W_vecmix_1,
           W_out_inv, b_out_inv, W_out_vec):
    x = x.astype(jnp.float32)
    xT = x.T
    xa = jnp.concatenate([x, jnp.ones((N, 1), jnp.float32)], axis=1)
    h0 = species_embed[h]
    hv0 = jnp.zeros((3, N, N_VEC), jnp.float32)

    def layer_args(W_r1, b_r1, W_r2, b_r2, W_r3, W_upd, b_upd, W_vecmix):
        return (W_r1, b_r1.reshape(1, -1), W_r2, b_r2.reshape(1, -1),
                W_r3[:, :D_INV], W_r3[:, D_INV:],
                W_upd[:D_INV], W_upd[D_INV:2 * D_INV], W_upd[2 * D_INV:],
                b_upd.reshape(1, -1),
                W_vecmix[:N_VEC], W_vecmix[N_VEC:])

    h1, hv1 = _layer_call(False)(
        x, xT, xa, h0, h0, hv0,
        *layer_args(W_r1_0, b_r1_0, W_r2_0, b_r2_0, W_r3_0, W_upd_0, b_upd_0, W_vecmix_0))

    invf, vfT = _layer_call(True)(
        x, xT, xa, h1, h1, hv1,
        *layer_args(W_r1_1, b_r1_1, W_r2_1, b_r2_1, W_r3_1, W_upd_1, b_upd_1, W_vecmix_1),
        W_out_inv, b_out_inv.reshape(1, -1), W_out_vec)

    vector_features = jnp.transpose(vfT, (1, 2, 0))
    return vector_features, invf
